# baseline (speedup 1.0000x reference)
import functools

import jax
import jax.numpy as jnp
from jax import lax
from jax.experimental import pallas as pl
from jax.experimental.pallas import tpu as pltpu
from jax.experimental.pallas import tpu_sc as plsc

EMBED_DIM = 64
ROW_PITCH = 128
NUM_WORKERS = 32
CHUNK = 128
NBUF = 6


@functools.partial(jax.jit, static_argnames=("n_rows",))
def _embed_gather(idx, table128, *, n_rows):
    per_w = n_rows // NUM_WORKERS
    cpw = per_w // CHUNK
    mesh = plsc.VectorSubcoreMesh(core_axis_name="c", subcore_axis_name="s")

    @functools.partial(
        pl.kernel,
        mesh=mesh,
        compiler_params=pltpu.CompilerParams(use_tc_tiling_on_sc=True),
        out_type=jax.ShapeDtypeStruct((n_rows, ROW_PITCH), jnp.float32),
        scratch_types=[
            pltpu.VMEM((per_w,), jnp.int32),
            pltpu.VMEM((NBUF, CHUNK, ROW_PITCH), jnp.float32),
            pltpu.SemaphoreType.DMA((NBUF,)),
            pltpu.SemaphoreType.DMA((NBUF,)),
        ],
    )
    def k(idx_hbm, table_hbm, out_hbm, idx_v, rows_v, gsem, osem):
        wid = lax.axis_index("s") * 2 + lax.axis_index("c")
        base = wid * per_w
        pltpu.sync_copy(idx_hbm.at[pl.ds(base, per_w)], idx_v)

        def gather_start(j, b):
            pltpu.async_copy(
                table_hbm.at[idx_v.at[pl.ds(j * CHUNK, CHUNK)]],
                rows_v.at[b],
                gsem.at[b],
            )

        def gather_wait(b):
            pltpu.make_async_copy(
                table_hbm.at[pl.ds(0, CHUNK)], rows_v.at[b], gsem.at[b]
            ).wait()

        def out_start(j, b):
            pltpu.async_copy(
                rows_v.at[b],
                out_hbm.at[pl.ds(base + j * CHUNK, CHUNK)],
                osem.at[b],
            )

        def out_wait(b):
            pltpu.make_async_copy(
                rows_v.at[b], out_hbm.at[pl.ds(base, CHUNK)], osem.at[b]
            ).wait()

        for b in range(min(NBUF, cpw)):
            gather_start(b, b)
        for j in range(cpw):
            b = j % NBUF
            gather_wait(b)
            out_start(j, b)
            if j + NBUF < cpw:
                out_wait(b)
                gather_start(j + NBUF, b)
        for j in range(max(cpw - NBUF, 0), cpw):
            out_wait(j % NBUF)

    return k(idx, table128)


def kernel(tokens, emb_weight):
    n = tokens.shape[0]
    n_pad = -(-n // (NUM_WORKERS * CHUNK)) * NUM_WORKERS * CHUNK
    table128 = jnp.pad(emb_weight, ((0, 0), (0, ROW_PITCH - EMBED_DIM)))
    idx = jnp.pad(tokens.astype(jnp.int32), (0, n_pad - n))
    out128 = _embed_gather(idx, table128, n_rows=n_pad)
    return out128[:n, :EMBED_DIM]


# speedup vs baseline: 1.0581x; 1.0581x over previous
import functools

import jax
import jax.numpy as jnp
from jax import lax
from jax.experimental import pallas as pl
from jax.experimental.pallas import tpu as pltpu
from jax.experimental.pallas import tpu_sc as plsc

EMBED_DIM = 64
NUM_WORKERS = 32
CHUNK = 128
NBUF = 12


@functools.partial(jax.jit, static_argnames=("n_rows",))
def _embed_gather(idx2d, table, *, n_rows):
    per_w = n_rows // NUM_WORKERS
    cpw = idx2d.shape[0] // NUM_WORKERS
    tail = per_w - (cpw - 1) * CHUNK
    mesh = plsc.VectorSubcoreMesh(core_axis_name="c", subcore_axis_name="s")

    @functools.partial(
        pl.kernel,
        mesh=mesh,
        compiler_params=pltpu.CompilerParams(use_tc_tiling_on_sc=False),
        out_type=jax.ShapeDtypeStruct((n_rows, EMBED_DIM), jnp.float32),
        scratch_types=[
            pltpu.VMEM((cpw, CHUNK), jnp.int32),
            pltpu.VMEM((NBUF, CHUNK, EMBED_DIM), jnp.float32),
            pltpu.SemaphoreType.DMA((NBUF,)),
            pltpu.SemaphoreType.DMA((NBUF,)),
        ],
    )
    def k(idx_hbm, table_hbm, out_hbm, idx_v, rows_v, gsem, osem):
        wid = lax.axis_index("s") * 2 + lax.axis_index("c")
        base = wid * per_w
        pltpu.sync_copy(idx_hbm.at[pl.ds(wid * cpw, cpw)], idx_v)

        def gather_start(j, b):
            pltpu.async_copy(
                table_hbm.at[idx_v.at[j]], rows_v.at[b], gsem.at[b]
            )

        def gather_wait(b):
            pltpu.make_async_copy(
                table_hbm.at[pl.ds(0, CHUNK)], rows_v.at[b], gsem.at[b]
            ).wait()

        def out_start(j, b, width):
            pltpu.async_copy(
                rows_v.at[b, pl.ds(0, width)],
                out_hbm.at[pl.ds(base + j * CHUNK, width)],
                osem.at[b],
            )

        def out_wait(b, width):
            pltpu.make_async_copy(
                rows_v.at[b, pl.ds(0, width)],
                out_hbm.at[pl.ds(base, width)],
                osem.at[b],
            ).wait()

        for b in range(min(NBUF, cpw)):
            gather_start(b, b)
        for j in range(cpw):
            b = j % NBUF
            width = CHUNK if j < cpw - 1 else tail
            gather_wait(b)
            out_start(j, b, width)
            if j + NBUF < cpw:
                out_wait(b, CHUNK)
                gather_start(j + NBUF, b)
        for j in range(max(cpw - NBUF, 0), cpw):
            width = CHUNK if j < cpw - 1 else tail
            out_wait(j % NBUF, width)

    return k(idx2d, table)


def kernel(tokens, emb_weight):
    n = tokens.shape[0]
    per_w = n // NUM_WORKERS
    assert per_w * NUM_WORKERS == n
    cpw = -(-per_w // CHUNK)
    idx = tokens.astype(jnp.int32).reshape(NUM_WORKERS, per_w)
    idx = jnp.pad(idx, ((0, 0), (0, cpw * CHUNK - per_w)))
    idx2d = idx.reshape(NUM_WORKERS * cpw, CHUNK)
    return _embed_gather(idx2d, emb_weight, n_rows=n)
